# Initial kernel scaffold; baseline (speedup 1.0000x reference)
import jax
import jax.numpy as jnp
from jax.experimental import pallas as pl


def kernel(flow0, flow1, flow2, flow3, xs, ys, ts, ps, frame, frame_, model_param):
    raise NotImplementedError("write your pallas kernel here")


# speedup vs baseline: 52.5441x; 52.5441x over previous
import functools

import jax
import jax.numpy as jnp
from jax import lax
from jax.experimental import pallas as pl
from jax.experimental.pallas import tpu as pltpu
from jax.experimental.pallas import tpu_sc as plsc

F32EPS = 1.1920928955078125e-07
B = 8
N = 100000
NW = 32
CHUNK = 2000
MICRO = 80
GRID = 87040
LEVELS012 = ((3, 32, 1024, 0), (2, 64, 4096, 1024), (1, 128, 16384, 5120))
L3_OFF = 21504


def _tc_prepass(ts_ref, ps_ref, f0_ref, f1_ref, f2_ref, f3_ref, mp_ref,
                tref_ref, pre_ref):
  t = ts_ref[...]
  p = ps_ref[...]
  i = lax.broadcasted_iota(jnp.int32, (B, N), 1)

  def t_at_last(mask):
    idx = jnp.max(jnp.where(mask, i, -1), axis=1)
    idx = jnp.where(idx < 0, N - 1, idx)
    return jnp.sum(jnp.where(i == idx[:, None], t, 0.0), axis=1)

  def t_at_first(mask):
    idx = jnp.min(jnp.where(mask, i, N), axis=1)
    idx = jnp.where(idx >= N, 0, idx)
    return jnp.sum(jnp.where(i == idx[:, None], t, 0.0), axis=1)

  pos = p > 0.0
  neg = p < 0.0
  tref_ref[...] = jnp.stack(
      [t_at_last(pos) + F32EPS, t_at_first(pos) - F32EPS,
       t_at_last(neg) + F32EPS, t_at_first(neg) - F32EPS], axis=1)

  def charb(d):
    return jnp.mean(jnp.exp(0.45 * jnp.log(d * d + 1e-6)))

  smooth = 0.0
  for ref in (f0_ref, f1_ref, f2_ref, f3_ref):
    f = ref[...]
    s = (charb(f[..., :, 1:] - f[..., :, :-1])
         + charb(f[..., 1:, :] - f[..., :-1, :])
         + charb(f[..., 1:, 1:] - f[..., :-1, :-1])
         + charb(f[..., :-1, 1:] - f[..., 1:, :-1]))
    smooth = smooth + s / 4.0

  mp = mp_ref[...]
  wd = jnp.sum(mp * mp) * (0.5 * 1e-4)
  pre_ref[...] = jnp.reshape(wd + smooth * 25.0, (1, 1))


def _sc_event_kernel(ev_hbm, fl0_hbm, fl1_hbm, fl2_hbm, f3rows_hbm, tref_hbm,
                     nums_hbm, dens_hbm,
                     ev_v, flow_v, num_v, den_v, tref_v, idx_v, rows_v, sem):
  cid = lax.axis_index("c")
  sid = lax.axis_index("s")
  wid = sid * 2 + cid
  b = wid // 4
  combo = wid % 4
  eb = b * N
  obase = wid * GRID
  lane = lax.iota(jnp.int32, 16)
  zi = jnp.zeros((16,), jnp.int32)
  zf = jnp.zeros((16,), jnp.float32)
  onef = zf + 1.0
  pol = jnp.where(combo < 2, 1.0, -1.0)

  pltpu.sync_copy(tref_hbm, tref_v)
  trefx = plsc.load_gather(tref_v, [zi + wid])

  def zero_grids(nwords):
    def zbody(k, _):
      num_v[pl.ds(k * 16, 16)] = zf
      den_v[pl.ds(k * 16, 16)] = zf
      return 0
    lax.fori_loop(0, nwords // 16, zbody, 0)

  def load_event(r):
    xf = plsc.load_gather(ev_v, [r, zi])
    yf = plsc.load_gather(ev_v, [r, zi + 1])
    t = plsc.load_gather(ev_v, [r, zi + 2])
    p = plsc.load_gather(ev_v, [r, zi + 3])
    return xf, yf, t, p

  def splat(xl, yl, t, m, fx, fy, w, num_mask_idx):
    t_ = trefx - t
    xw = jnp.clip(xl + t_ * fx, 0.0, w - 1.0)
    yw = jnp.clip(yl + t_ * fy, 0.0, w - 1.0)
    x0i = xw.astype(jnp.int32)
    y0i = yw.astype(jnp.int32)
    ax = xw - x0i.astype(jnp.float32)
    bx = 1.0 - ax
    ay = yw - y0i.astype(jnp.float32)
    by = 1.0 - ay
    x1i = jnp.minimum(x0i + 1, w - 1)
    y1i = jnp.minimum(y0i + 1, w - 1)
    if num_mask_idx is None:
      m0 = m
      m1 = m
      conv = lambda v: v
    else:
      hp, mask_all = num_mask_idx
      m0 = m & ((y0i >> 7) == hp)
      m1 = m & ((y1i >> 7) == hp)
      conv = lambda v: v & mask_all
    row0 = y0i * w
    row1 = y1i * w
    ia = conv(x1i + row1)
    ib = conv(x0i + row1)
    ic = conv(x1i + row0)
    id_ = conv(x0i + row0)
    plsc.addupdate_scatter(den_v, [ia], onef, mask=m1)
    plsc.addupdate_scatter(den_v, [ib], onef, mask=m1)
    plsc.addupdate_scatter(den_v, [ic], onef, mask=m0)
    plsc.addupdate_scatter(den_v, [id_], onef, mask=m0)
    plsc.addupdate_scatter(num_v, [ia], ax * ay * t, mask=m1)
    plsc.addupdate_scatter(num_v, [ib], bx * ay * t, mask=m1)
    plsc.addupdate_scatter(num_v, [ic], ax * by * t, mask=m0)
    plsc.addupdate_scatter(num_v, [id_], bx * by * t, mask=m0)

  for (shift, w, hw, ooff), fl_hbm in zip(LEVELS012, (fl0_hbm, fl1_hbm, fl2_hbm)):
    pltpu.sync_copy(fl_hbm.at[pl.ds(b * 2 * hw, 2 * hw)],
                    flow_v.at[pl.ds(0, 2 * hw)])
    zero_grids(hw)

    def cbody(ci, _, shift=shift, w=w, hw=hw):
      pltpu.sync_copy(ev_hbm.at[pl.ds(eb + ci * CHUNK, CHUNK)], ev_v)

      def vbody(k, _):
        r = k * 16 + lane
        xf, yf, t, p = load_event(r)
        m = p == pol
        xi = xf.astype(jnp.int32) >> shift
        yi = yf.astype(jnp.int32) >> shift
        pix = yi * w + xi
        fx = plsc.load_gather(flow_v, [pix])
        fy = plsc.load_gather(flow_v, [pix + hw])
        splat(xi.astype(jnp.float32), yi.astype(jnp.float32), t, m,
              fx, fy, w, None)
        return 0

      lax.fori_loop(0, CHUNK // 16, vbody, 0)
      return 0

    lax.fori_loop(0, N // CHUNK, cbody, 0)
    pltpu.sync_copy(num_v.at[pl.ds(0, hw)],
                    nums_hbm.at[pl.ds(obase + ooff, hw)])
    pltpu.sync_copy(den_v.at[pl.ds(0, hw)],
                    dens_hbm.at[pl.ds(obase + ooff, hw)])

  for hp in range(2):
    zero_grids(32768)

    def cbody3(ci, _, hp=hp):
      pltpu.sync_copy(ev_hbm.at[pl.ds(eb + ci * CHUNK, CHUNK)], ev_v)

      def mbody(mi, _):
        mb = mi * MICRO
        for k in range(MICRO // 16):
          r = mb + k * 16 + lane
          xf = plsc.load_gather(ev_v, [r, zi])
          yf = plsc.load_gather(ev_v, [r, zi + 1])
          pix = yf.astype(jnp.int32) * 256 + xf.astype(jnp.int32)
          idx_v[pl.ds(k * 16, 16)] = pix + b * 65536
        pltpu.async_copy(f3rows_hbm.at[idx_v], rows_v, sem).wait()
        for k in range(MICRO // 16):
          r = mb + k * 16 + lane
          rr = k * 16 + lane
          xf, yf, t, p = load_event(r)
          m = p == pol
          fx = plsc.load_gather(rows_v, [rr, zi])
          fy = plsc.load_gather(rows_v, [rr, zi + 1])
          splat(xf, yf, t, m, fx, fy, 256, (hp, 32767))
        return 0

      lax.fori_loop(0, CHUNK // MICRO, mbody, 0)
      return 0

    lax.fori_loop(0, N // CHUNK, cbody3, 0)
    pltpu.sync_copy(num_v.at[pl.ds(0, 32768)],
                    nums_hbm.at[pl.ds(obase + L3_OFF + hp * 32768, 32768)])
    pltpu.sync_copy(den_v.at[pl.ds(0, 32768)],
                    dens_hbm.at[pl.ds(obase + L3_OFF + hp * 32768, 32768)])


def _tc_postpass(nums_ref, dens_ref, pre_ref, out_ref):
  n = nums_ref[...]
  d = dens_ref[...]
  g = n / (d + F32EPS)
  out_ref[...] = pre_ref[...] + jnp.reshape(jnp.sum(jnp.sqrt(g * g + 1e-6)),
                                            (1, 1))


@jax.jit
def kernel(flow0, flow1, flow2, flow3, xs, ys, ts, ps, frame, frame_,
           model_param):
  ps_f = ps.astype(jnp.float32)
  tref, pre = pl.pallas_call(
      _tc_prepass,
      out_shape=(jax.ShapeDtypeStruct((B, 4), jnp.float32),
                 jax.ShapeDtypeStruct((1, 1), jnp.float32)),
  )(ts, ps_f, flow0, flow1, flow2, flow3, model_param.reshape(8192, 128))

  ev = jnp.stack(
      [xs.astype(jnp.float32), ys.astype(jnp.float32), ts, ps_f],
      axis=-1).reshape(B * N, 4)
  f3rows = jnp.transpose(flow3, (0, 2, 3, 1)).reshape(B * 65536, 2)

  mesh = plsc.VectorSubcoreMesh(core_axis_name="c", subcore_axis_name="s",
                                num_cores=2, num_subcores=16)
  nums, dens = pl.kernel(
      _sc_event_kernel,
      out_type=(jax.ShapeDtypeStruct((NW * GRID,), jnp.float32),
                jax.ShapeDtypeStruct((NW * GRID,), jnp.float32)),
      mesh=mesh,
      compiler_params=pltpu.CompilerParams(use_tc_tiling_on_sc=False, needs_layout_passes=False),
      scratch_types=[
          pltpu.VMEM((CHUNK, 4), jnp.float32),
          pltpu.VMEM((32768,), jnp.float32),
          pltpu.VMEM((32768,), jnp.float32),
          pltpu.VMEM((32768,), jnp.float32),
          pltpu.VMEM((32,), jnp.float32),
          pltpu.VMEM((MICRO,), jnp.int32),
          pltpu.VMEM((MICRO, 2), jnp.float32),
          pltpu.SemaphoreType.DMA,
      ],
  )(ev, flow0.reshape(-1), flow1.reshape(-1), flow2.reshape(-1), f3rows,
    tref.reshape(NW))

  out = pl.pallas_call(
      _tc_postpass,
      out_shape=jax.ShapeDtypeStruct((1, 1), jnp.float32),
  )(nums.reshape(NW * GRID // 128, 128), dens.reshape(NW * GRID // 128, 128),
    pre)
  return out[0, 0]


# speedup vs baseline: 66.7740x; 1.2708x over previous
import functools

import jax
import jax.numpy as jnp
from jax import lax
from jax.experimental import pallas as pl
from jax.experimental.pallas import tpu as pltpu
from jax.experimental.pallas import tpu_sc as plsc

F32EPS = 1.1920928955078125e-07
B = 8
N = 100000
NW = 32
CHUNK = 2000
MICRO = 80
GRID = 87040
LEVELS012 = ((3, 32, 1024, 0), (2, 64, 4096, 1024), (1, 128, 16384, 5120))
L3_OFF = 21504


def _tc_prepass(ts_ref, ps_ref, f0_ref, f1_ref, f2_ref, f3_ref, mp_ref,
                tref_ref, pre_ref):
  t = ts_ref[...]
  p = ps_ref[...]
  i = lax.broadcasted_iota(jnp.int32, (B, N), 1)

  def t_at_last(mask):
    idx = jnp.max(jnp.where(mask, i, -1), axis=1)
    idx = jnp.where(idx < 0, N - 1, idx)
    return jnp.sum(jnp.where(i == idx[:, None], t, 0.0), axis=1)

  def t_at_first(mask):
    idx = jnp.min(jnp.where(mask, i, N), axis=1)
    idx = jnp.where(idx >= N, 0, idx)
    return jnp.sum(jnp.where(i == idx[:, None], t, 0.0), axis=1)

  pos = p > 0.0
  neg = p < 0.0
  tref_ref[...] = jnp.stack(
      [t_at_last(pos) + F32EPS, t_at_first(pos) - F32EPS,
       t_at_last(neg) + F32EPS, t_at_first(neg) - F32EPS], axis=1)

  def charb(d):
    return jnp.mean(jnp.exp(0.45 * jnp.log(d * d + 1e-6)))

  smooth = 0.0
  for ref in (f0_ref, f1_ref, f2_ref, f3_ref):
    f = ref[...]
    s = (charb(f[..., :, 1:] - f[..., :, :-1])
         + charb(f[..., 1:, :] - f[..., :-1, :])
         + charb(f[..., 1:, 1:] - f[..., :-1, :-1])
         + charb(f[..., :-1, 1:] - f[..., 1:, :-1]))
    smooth = smooth + s / 4.0

  mp = mp_ref[...]
  wd = jnp.sum(mp * mp) * (0.5 * 1e-4)
  pre_ref[...] = jnp.reshape(wd + smooth * 25.0, (1, 1))


def _sc_event_kernel(ev_hbm, fl0_hbm, fl1_hbm, fl2_hbm, f3rows_hbm, tref_hbm,
                     nums_hbm, dens_hbm,
                     ev_v, flow_v, num_v, den_v, tref_v, idx_v, rows_v, sem,
                     idx_b, rows_b, sem_b):
  cid = lax.axis_index("c")
  sid = lax.axis_index("s")
  wid = sid * 2 + cid
  b = wid // 4
  combo = wid % 4
  eb = b * N
  obase = wid * GRID
  lane = lax.iota(jnp.int32, 16)
  zi = jnp.zeros((16,), jnp.int32)
  zf = jnp.zeros((16,), jnp.float32)
  onef = zf + 1.0
  pol = jnp.where(combo < 2, 1.0, -1.0)

  pltpu.sync_copy(tref_hbm, tref_v)
  trefx = plsc.load_gather(tref_v, [zi + wid])

  def zero_grids(nwords):
    def zbody(k, _):
      num_v[pl.ds(k * 16, 16)] = zf
      den_v[pl.ds(k * 16, 16)] = zf
      return 0
    lax.fori_loop(0, nwords // 16, zbody, 0)

  def load_event(r):
    xf = plsc.load_gather(ev_v, [r, zi])
    yf = plsc.load_gather(ev_v, [r, zi + 1])
    t = plsc.load_gather(ev_v, [r, zi + 2])
    p = plsc.load_gather(ev_v, [r, zi + 3])
    return xf, yf, t, p

  def splat(xl, yl, t, m, fx, fy, w, num_mask_idx):
    t_ = trefx - t
    xw = jnp.clip(xl + t_ * fx, 0.0, w - 1.0)
    yw = jnp.clip(yl + t_ * fy, 0.0, w - 1.0)
    x0i = xw.astype(jnp.int32)
    y0i = yw.astype(jnp.int32)
    ax = xw - x0i.astype(jnp.float32)
    bx = 1.0 - ax
    ay = yw - y0i.astype(jnp.float32)
    by = 1.0 - ay
    x1i = jnp.minimum(x0i + 1, w - 1)
    y1i = jnp.minimum(y0i + 1, w - 1)
    if num_mask_idx is None:
      m0 = m
      m1 = m
      conv = lambda v: v
    else:
      hp, mask_all = num_mask_idx
      m0 = m & ((y0i >> 7) == hp)
      m1 = m & ((y1i >> 7) == hp)
      conv = lambda v: v & mask_all
    row0 = y0i * w
    row1 = y1i * w
    ia = conv(x1i + row1)
    ib = conv(x0i + row1)
    ic = conv(x1i + row0)
    id_ = conv(x0i + row0)
    plsc.addupdate_scatter(den_v, [ia], onef, mask=m1)
    plsc.addupdate_scatter(den_v, [ib], onef, mask=m1)
    plsc.addupdate_scatter(den_v, [ic], onef, mask=m0)
    plsc.addupdate_scatter(den_v, [id_], onef, mask=m0)
    plsc.addupdate_scatter(num_v, [ia], ax * ay * t, mask=m1)
    plsc.addupdate_scatter(num_v, [ib], bx * ay * t, mask=m1)
    plsc.addupdate_scatter(num_v, [ic], ax * by * t, mask=m0)
    plsc.addupdate_scatter(num_v, [id_], bx * by * t, mask=m0)

  for (shift, w, hw, ooff), fl_hbm in zip(LEVELS012, (fl0_hbm, fl1_hbm, fl2_hbm)):
    pltpu.sync_copy(fl_hbm.at[pl.ds(b * 2 * hw, 2 * hw)],
                    flow_v.at[pl.ds(0, 2 * hw)])
    zero_grids(hw)

    def cbody(ci, _, shift=shift, w=w, hw=hw):
      pltpu.sync_copy(ev_hbm.at[pl.ds(eb + ci * CHUNK, CHUNK)], ev_v)

      def vbody(k, _):
        r = k * 16 + lane
        xf, yf, t, p = load_event(r)
        m = p == pol
        xi = xf.astype(jnp.int32) >> shift
        yi = yf.astype(jnp.int32) >> shift
        pix = yi * w + xi
        fx = plsc.load_gather(flow_v, [pix])
        fy = plsc.load_gather(flow_v, [pix + hw])
        splat(xi.astype(jnp.float32), yi.astype(jnp.float32), t, m,
              fx, fy, w, None)
        return 0

      lax.fori_loop(0, CHUNK // 16, vbody, 0)
      return 0

    lax.fori_loop(0, N // CHUNK, cbody, 0)
    pltpu.sync_copy(num_v.at[pl.ds(0, hw)],
                    nums_hbm.at[pl.ds(obase + ooff, hw)])
    pltpu.sync_copy(den_v.at[pl.ds(0, hw)],
                    dens_hbm.at[pl.ds(obase + ooff, hw)])

  def calc_idx(mi, ibuf):
    mb = mi * MICRO
    for k in range(MICRO // 16):
      r = mb + k * 16 + lane
      xf = plsc.load_gather(ev_v, [r, zi])
      yf = plsc.load_gather(ev_v, [r, zi + 1])
      pix = yf.astype(jnp.int32) * 256 + xf.astype(jnp.int32)
      ibuf[pl.ds(k * 16, 16)] = pix + b * 65536

  for hp in range(2):
    zero_grids(32768)

    def proc(mi, rbuf, hp=hp):
      mb = mi * MICRO
      for k in range(MICRO // 16):
        r = mb + k * 16 + lane
        rr = k * 16 + lane
        xf, yf, t, p = load_event(r)
        m = p == pol
        fx = plsc.load_gather(rbuf, [rr, zi])
        fy = plsc.load_gather(rbuf, [rr, zi + 1])
        splat(xf, yf, t, m, fx, fy, 256, (hp, 32767))

    def cbody3(ci, _, proc=proc):
      pltpu.sync_copy(ev_hbm.at[pl.ds(eb + ci * CHUNK, CHUNK)], ev_v)
      calc_idx(0, idx_v)
      pltpu.async_copy(f3rows_hbm.at[idx_v], rows_v, sem)

      def pbody(s, _):
        calc_idx(2 * s + 1, idx_b)
        pltpu.async_copy(f3rows_hbm.at[idx_b], rows_b, sem_b)
        pltpu.make_async_copy(f3rows_hbm.at[idx_v], rows_v, sem).wait()
        proc(2 * s, rows_v)
        calc_idx(2 * s + 2, idx_v)
        pltpu.async_copy(f3rows_hbm.at[idx_v], rows_v, sem)
        pltpu.make_async_copy(f3rows_hbm.at[idx_b], rows_b, sem_b).wait()
        proc(2 * s + 1, rows_b)
        return 0

      lax.fori_loop(0, (CHUNK // MICRO - 1) // 2, pbody, 0)
      pltpu.make_async_copy(f3rows_hbm.at[idx_v], rows_v, sem).wait()
      proc(CHUNK // MICRO - 1, rows_v)
      return 0

    lax.fori_loop(0, N // CHUNK, cbody3, 0)
    pltpu.sync_copy(num_v.at[pl.ds(0, 32768)],
                    nums_hbm.at[pl.ds(obase + L3_OFF + hp * 32768, 32768)])
    pltpu.sync_copy(den_v.at[pl.ds(0, 32768)],
                    dens_hbm.at[pl.ds(obase + L3_OFF + hp * 32768, 32768)])


def _tc_postpass(nums_ref, dens_ref, pre_ref, out_ref):
  n = nums_ref[...]
  d = dens_ref[...]
  g = n / (d + F32EPS)
  out_ref[...] = pre_ref[...] + jnp.reshape(jnp.sum(jnp.sqrt(g * g + 1e-6)),
                                            (1, 1))


@jax.jit
def kernel(flow0, flow1, flow2, flow3, xs, ys, ts, ps, frame, frame_,
           model_param):
  ps_f = ps.astype(jnp.float32)
  tref, pre = pl.pallas_call(
      _tc_prepass,
      out_shape=(jax.ShapeDtypeStruct((B, 4), jnp.float32),
                 jax.ShapeDtypeStruct((1, 1), jnp.float32)),
  )(ts, ps_f, flow0, flow1, flow2, flow3, model_param.reshape(8192, 128))

  ev = jnp.stack(
      [xs.astype(jnp.float32), ys.astype(jnp.float32), ts, ps_f],
      axis=-1).reshape(B * N, 4)
  f3rows = jnp.transpose(flow3, (0, 2, 3, 1)).reshape(B * 65536, 2)

  mesh = plsc.VectorSubcoreMesh(core_axis_name="c", subcore_axis_name="s",
                                num_cores=2, num_subcores=16)
  nums, dens = pl.kernel(
      _sc_event_kernel,
      out_type=(jax.ShapeDtypeStruct((NW * GRID,), jnp.float32),
                jax.ShapeDtypeStruct((NW * GRID,), jnp.float32)),
      mesh=mesh,
      compiler_params=pltpu.CompilerParams(use_tc_tiling_on_sc=False, needs_layout_passes=False),
      scratch_types=[
          pltpu.VMEM((CHUNK, 4), jnp.float32),
          pltpu.VMEM((32768,), jnp.float32),
          pltpu.VMEM((32768,), jnp.float32),
          pltpu.VMEM((32768,), jnp.float32),
          pltpu.VMEM((32,), jnp.float32),
          pltpu.VMEM((MICRO,), jnp.int32),
          pltpu.VMEM((MICRO, 2), jnp.float32),
          pltpu.SemaphoreType.DMA,
          pltpu.VMEM((MICRO,), jnp.int32),
          pltpu.VMEM((MICRO, 2), jnp.float32),
          pltpu.SemaphoreType.DMA,
      ],
  )(ev, flow0.reshape(-1), flow1.reshape(-1), flow2.reshape(-1), f3rows,
    tref.reshape(NW))

  out = pl.pallas_call(
      _tc_postpass,
      out_shape=jax.ShapeDtypeStruct((1, 1), jnp.float32),
  )(nums.reshape(NW * GRID // 128, 128), dens.reshape(NW * GRID // 128, 128),
    pre)
  return out[0, 0]
